# baseline (speedup 1.0000x reference)
import functools

import jax
import jax.numpy as jnp
from jax import lax
from jax.experimental import pallas as pl
from jax.experimental.pallas import tpu as pltpu
from jax.experimental.pallas import tpu_sc as plsc

C = 16384
A = 200
NC = 2
NS = 16
NW = NC * NS
R = C // NW
CR = 64
NCH = R // CR
PAIRS = CR // 2
SLABS = 2 * A // 16

_mesh = plsc.VectorSubcoreMesh(core_axis_name="c", subcore_axis_name="s")


@functools.partial(
    pl.kernel,
    mesh=_mesh,
    out_type=jax.ShapeDtypeStruct((C,), jnp.float32),
    compiler_params=pltpu.CompilerParams(needs_layout_passes=False),
    scratch_types=[
        pltpu.VMEM((CR * A,), jnp.int32),
        pltpu.VMEM((CR,), jnp.float32),
        pltpu.VMEM((CR,), jnp.float32),
        pltpu.VMEM((CR * 16,), jnp.float32),
        pltpu.VMEM((128,), jnp.float32),
        pltpu.VMEM((16,), jnp.float32),
    ],
)
def _sc_energy_adder(species_hbm, energies_hbm, table_hbm, icpt_hbm,
                     out_hbm, sp_v, en_v, out_v, psum_v, tab_v, icpt_s):
    wid = lax.axis_index("s") * NC + lax.axis_index("c")
    row0 = wid * R

    pltpu.sync_copy(table_hbm, tab_v.at[pl.ds(0, 4)])
    pltpu.sync_copy(icpt_hbm, icpt_s)
    icpt = icpt_s[...]
    lane = lax.iota(jnp.int32, 16)
    lo_mask = lane < 8
    zero = jnp.zeros((16,), jnp.float32)

    def chunk_body(cidx, _):
        r0 = row0 + cidx * CR
        pltpu.sync_copy(species_hbm.at[pl.ds(r0 * A, CR * A)], sp_v)
        pltpu.sync_copy(energies_hbm.at[pl.ds(r0, CR)], en_v)

        def pair_body(p, _):
            base = p * (2 * A)
            acc_a = zero
            for j in range(12):
                s = sp_v[pl.ds(base + j * 16, 16)]
                acc_a = acc_a + plsc.load_gather(tab_v, [s])
            s = sp_v[pl.ds(base + 192, 16)]
            v = plsc.load_gather(tab_v, [s])
            acc_a = acc_a + jnp.where(lo_mask, v, zero)
            acc_b = jnp.where(lo_mask, zero, v)
            for j in range(13, SLABS):
                s = sp_v[pl.ds(base + j * 16, 16)]
                acc_b = acc_b + plsc.load_gather(tab_v, [s])
            psum_v[pl.ds(2 * p * 16, 16)] = acc_a
            psum_v[pl.ds((2 * p + 1) * 16, 16)] = acc_b
            return _

        lax.fori_loop(0, PAIRS, pair_body, None)

        for g in range(CR // 16):
            tot = jnp.full((16,), 0.0, jnp.float32)
            col0 = g * 256 + lane * 16
            for j in range(16):
                tot = tot + plsc.load_gather(psum_v, [col0 + j])
            sl = pl.ds(g * 16, 16)
            out_v[sl] = tot + en_v[sl] + icpt
        pltpu.sync_copy(out_v, out_hbm.at[pl.ds(r0, CR)])
        return _

    lax.fori_loop(0, NCH, chunk_body, None)


def kernel(species, energies, self_energies, intercept):
    sae = _sc_energy_adder(
        species.reshape(-1),
        energies,
        self_energies,
        jnp.broadcast_to(intercept, (16,)),
    )
    return (species, sae)
